# baseline (speedup 1.0000x reference)
import functools

import jax
import jax.numpy as jnp
from jax import lax
from jax.experimental import pallas as pl
from jax.experimental.pallas import tpu as pltpu
from jax.experimental.pallas import tpu_sc as plsc


_TN = 512
_TCB = 1024


def _argmax_body(x_ref, e_ref, idx_ref, maxv, maxi):
    j = pl.program_id(1)

    xb = x_ref[...]
    xn = xb / jnp.clip(
        jnp.sqrt(jnp.sum(xb * xb, axis=-1, keepdims=True)), 1e-12, None)
    eb = e_ref[...]
    en = eb / jnp.clip(
        jnp.sqrt(jnp.sum(eb * eb, axis=-1, keepdims=True)), 1e-12, None)

    dist = lax.dot_general(
        xn, en, (((1,), (1,)), ((), ())),
        preferred_element_type=jnp.float32)

    bmax = jnp.max(dist, axis=-1)
    iota = lax.broadcasted_iota(jnp.int32, dist.shape, 1)
    bidx = jnp.min(
        jnp.where(dist == bmax[:, None], iota, jnp.int32(2**30)),
        axis=-1) + j * _TCB

    @pl.when(j == 0)
    def _():
        maxv[...] = bmax
        maxi[...] = bidx

    @pl.when(j > 0)
    def _():
        better = bmax > maxv[...]
        maxv[...] = jnp.where(better, bmax, maxv[...])
        maxi[...] = jnp.where(better, bidx, maxi[...])

    @pl.when(j == pl.num_programs(1) - 1)
    def _():
        idx_ref[...] = maxi[...].reshape(idx_ref.shape)


def _vq_argmax(x_flat, e):
    n, d = x_flat.shape
    c = e.shape[0]
    nb, ncb = n // _TN, c // _TCB
    out = pl.pallas_call(
        _argmax_body,
        grid=(nb, ncb),
        in_specs=[
            pl.BlockSpec((_TN, d), lambda i, j: (i, 0)),
            pl.BlockSpec((_TCB, d), lambda i, j: (j, 0)),
        ],
        out_specs=pl.BlockSpec((1, 1, _TN), lambda i, j: (i, 0, 0)),
        out_shape=jax.ShapeDtypeStruct((nb, 1, _TN), jnp.int32),
        scratch_shapes=[
            pltpu.VMEM((_TN,), jnp.float32),
            pltpu.VMEM((_TN,), jnp.int32),
        ],
        compiler_params=pltpu.CompilerParams(
            dimension_semantics=("parallel", "arbitrary")),
    )(x_flat, e)
    return out.reshape(n)


_NW = 32
_CH = 128


def _make_sc_gather(b, v, d):
    b_per_w = b // _NW
    n_ch = b_per_w // _CH
    mesh = plsc.VectorSubcoreMesh(core_axis_name="c", subcore_axis_name="s")

    @functools.partial(
        pl.kernel, mesh=mesh,
        out_type=jax.ShapeDtypeStruct((b, d), jnp.float32),
        scratch_types=[
            pltpu.VMEM((n_ch, _CH), jnp.int32),
            pltpu.VMEM((b_per_w, d), jnp.float32),
            pltpu.SemaphoreType.DMA,
        ],
        compiler_params=pltpu.CompilerParams(use_tc_tiling_on_sc=False),
    )
    def gather_k(idx_hbm, table_hbm, out_hbm, idx_v, rows_v, sem):
        wid = lax.axis_index("s") * 2 + lax.axis_index("c")
        base = wid * b_per_w
        pltpu.sync_copy(idx_hbm.at[wid], idx_v)
        copies = []
        for k in range(n_ch):
            copies.append(pltpu.async_copy(
                table_hbm.at[idx_v.at[k]],
                rows_v.at[pl.ds(k * _CH, _CH)], sem))
        for cp in copies:
            cp.wait()
        pltpu.sync_copy(rows_v, out_hbm.at[pl.ds(base, b_per_w)])

    return gather_k


def kernel(x, embed):
    xf = x.astype(jnp.float32)
    b0, b1, d = xf.shape
    c = embed.shape[1]
    n = b0 * b1

    x_flat = xf.reshape(n, d)
    e = embed.reshape(c, d)

    idx = _vq_argmax(x_flat, e)

    idx3 = idx.reshape(_NW, (n // _NW) // _CH, _CH)
    quant = _make_sc_gather(n, c, d)(idx3, e)

    return quant.reshape(b0, b1, d), idx.reshape(b0, b1)


# speedup vs baseline: 1.5249x; 1.5249x over previous
import functools

import jax
import jax.numpy as jnp
from jax import lax
from jax.experimental import pallas as pl
from jax.experimental.pallas import tpu as pltpu
from jax.experimental.pallas import tpu_sc as plsc


_TN = 512
_TCB = 1024


def _argmax_body(x_ref, e_ref, idx_ref, xn_s, en_s, runmax, runidx):
    j = pl.program_id(1)
    nj = pl.num_programs(1)

    @pl.when((pl.program_id(0) == 0) & (j == 0))
    def _():
        e = e_ref[...]
        en_s[...] = e / jnp.clip(
            jnp.sqrt(jnp.sum(e * e, axis=-1, keepdims=True)), 1e-12, None)

    @pl.when(j == 0)
    def _():
        xb = x_ref[...]
        xn_s[...] = xb / jnp.clip(
            jnp.sqrt(jnp.sum(xb * xb, axis=-1, keepdims=True)), 1e-12, None)

    xn = xn_s[...]
    en = en_s[pl.ds(j * _TCB, _TCB), :]
    dist = lax.dot_general(
        xn, en, (((1,), (1,)), ((), ())),
        preferred_element_type=jnp.float32)
    cidx = lax.broadcasted_iota(jnp.int32, dist.shape, 1) + j * _TCB

    @pl.when(j == 0)
    def _():
        runmax[...] = dist
        runidx[...] = cidx

    @pl.when(j > 0)
    def _():
        better = dist > runmax[...]
        runmax[...] = jnp.maximum(dist, runmax[...])
        runidx[...] = jnp.where(better, cidx, runidx[...])

    @pl.when(j == nj - 1)
    def _():
        rm = runmax[...]
        rowmax = jnp.max(rm, axis=-1)
        cand = jnp.where(rm == rowmax[:, None], runidx[...], jnp.int32(2**30))
        idx_ref[...] = jnp.min(cand, axis=-1).reshape(idx_ref.shape)


def _vq_argmax(x_flat, e):
    n, d = x_flat.shape
    c = e.shape[0]
    nb, ncb = n // _TN, c // _TCB
    out = pl.pallas_call(
        _argmax_body,
        grid=(nb, ncb),
        in_specs=[
            pl.BlockSpec((_TN, d), lambda i, j: (i, 0)),
            pl.BlockSpec((c, d), lambda i, j: (0, 0)),
        ],
        out_specs=pl.BlockSpec((1, 1, _TN), lambda i, j: (i, 0, 0)),
        out_shape=jax.ShapeDtypeStruct((nb, 1, _TN), jnp.int32),
        scratch_shapes=[
            pltpu.VMEM((_TN, d), jnp.float32),
            pltpu.VMEM((c, d), jnp.float32),
            pltpu.VMEM((_TN, _TCB), jnp.float32),
            pltpu.VMEM((_TN, _TCB), jnp.int32),
        ],
        compiler_params=pltpu.CompilerParams(
            dimension_semantics=("parallel", "arbitrary")),
    )(x_flat, e)
    return out.reshape(n)


_NW = 32
_CH = 128


def _make_sc_gather(b, v, d):
    b_per_w = b // _NW
    n_ch = b_per_w // _CH
    mesh = plsc.VectorSubcoreMesh(core_axis_name="c", subcore_axis_name="s")

    @functools.partial(
        pl.kernel, mesh=mesh,
        out_type=jax.ShapeDtypeStruct((b, d), jnp.float32),
        scratch_types=[
            pltpu.VMEM((n_ch, _CH), jnp.int32),
            pltpu.VMEM((b_per_w, d), jnp.float32),
            pltpu.SemaphoreType.DMA,
        ],
        compiler_params=pltpu.CompilerParams(use_tc_tiling_on_sc=False),
    )
    def gather_k(idx_hbm, table_hbm, out_hbm, idx_v, rows_v, sem):
        wid = lax.axis_index("s") * 2 + lax.axis_index("c")
        base = wid * b_per_w
        pltpu.sync_copy(idx_hbm.at[wid], idx_v)
        copies = []
        for k in range(n_ch):
            copies.append(pltpu.async_copy(
                table_hbm.at[idx_v.at[k]],
                rows_v.at[pl.ds(k * _CH, _CH)], sem))
        for cp in copies:
            cp.wait()
        pltpu.sync_copy(rows_v, out_hbm.at[pl.ds(base, b_per_w)])

    return gather_k


def kernel(x, embed):
    xf = x.astype(jnp.float32)
    b0, b1, d = xf.shape
    c = embed.shape[1]
    n = b0 * b1

    x_flat = xf.reshape(n, d)
    e = embed.reshape(c, d)

    idx = _vq_argmax(x_flat, e)

    idx3 = idx.reshape(_NW, (n // _NW) // _CH, _CH)
    quant = _make_sc_gather(n, c, d)(idx3, e)

    return quant.reshape(b0, b1, d), idx.reshape(b0, b1)


# speedup vs baseline: 2.3105x; 1.5152x over previous
import functools

import jax
import jax.numpy as jnp
from jax import lax
from jax.experimental import pallas as pl
from jax.experimental.pallas import tpu as pltpu
from jax.experimental.pallas import tpu_sc as plsc


_TN = 512
_TCB = 1024


def _argmax_body(x_ref, e_ref, idx_ref, xn_s, en_s, runmax, runidx):
    j = pl.program_id(1)
    nj = pl.num_programs(1)

    @pl.when((pl.program_id(0) == 0) & (j == 0))
    def _():
        e = e_ref[...]
        en_s[...] = e / jnp.clip(
            jnp.sqrt(jnp.sum(e * e, axis=-1, keepdims=True)), 1e-12, None)

    @pl.when(j == 0)
    def _():
        xb = x_ref[...]
        xn_s[...] = xb / jnp.clip(
            jnp.sqrt(jnp.sum(xb * xb, axis=-1, keepdims=True)), 1e-12, None)

    xn = xn_s[...]
    en = en_s[pl.ds(j * _TCB, _TCB), :]
    dist = lax.dot_general(
        xn, en, (((1,), (1,)), ((), ())),
        preferred_element_type=jnp.float32)

    ng = _TCB // 128
    vals = [dist[:, s * 128:(s + 1) * 128] for s in range(ng)]
    idxs = [jnp.full((_TN, 128), s * 128, jnp.int32) for s in range(ng)]
    while len(vals) > 1:
        nv, ni = [], []
        for p in range(0, len(vals), 2):
            vl, vr = vals[p], vals[p + 1]
            il, ir = idxs[p], idxs[p + 1]
            take_r = vr > vl
            nv.append(jnp.maximum(vl, vr))
            ni.append(jnp.where(take_r, ir, il))
        vals, idxs = nv, ni
    tmax = vals[0]
    cidx = (j * _TCB + idxs[0]
            + lax.broadcasted_iota(jnp.int32, tmax.shape, 1))

    @pl.when(j == 0)
    def _():
        runmax[...] = tmax
        runidx[...] = cidx

    @pl.when(j > 0)
    def _():
        better = tmax > runmax[...]
        runmax[...] = jnp.maximum(tmax, runmax[...])
        runidx[...] = jnp.where(better, cidx, runidx[...])

    @pl.when(j == nj - 1)
    def _():
        rm = runmax[...]
        rowmax = jnp.max(rm, axis=-1)
        cand = jnp.where(rm == rowmax[:, None], runidx[...], jnp.int32(2**30))
        idx_ref[...] = jnp.min(cand, axis=-1).reshape(idx_ref.shape)


def _vq_argmax(x_flat, e):
    n, d = x_flat.shape
    c = e.shape[0]
    nb, ncb = n // _TN, c // _TCB
    out = pl.pallas_call(
        _argmax_body,
        grid=(nb, ncb),
        in_specs=[
            pl.BlockSpec((_TN, d), lambda i, j: (i, 0)),
            pl.BlockSpec((c, d), lambda i, j: (0, 0)),
        ],
        out_specs=pl.BlockSpec((1, 1, _TN), lambda i, j: (i, 0, 0)),
        out_shape=jax.ShapeDtypeStruct((nb, 1, _TN), jnp.int32),
        scratch_shapes=[
            pltpu.VMEM((_TN, d), jnp.float32),
            pltpu.VMEM((c, d), jnp.float32),
            pltpu.VMEM((_TN, 128), jnp.float32),
            pltpu.VMEM((_TN, 128), jnp.int32),
        ],
        compiler_params=pltpu.CompilerParams(
            dimension_semantics=("parallel", "arbitrary")),
    )(x_flat, e)
    return out.reshape(n)


_NW = 32
_CH = 128


def _make_sc_gather(b, v, d):
    b_per_w = b // _NW
    n_ch = b_per_w // _CH
    mesh = plsc.VectorSubcoreMesh(core_axis_name="c", subcore_axis_name="s")

    @functools.partial(
        pl.kernel, mesh=mesh,
        out_type=jax.ShapeDtypeStruct((b, d), jnp.float32),
        scratch_types=[
            pltpu.VMEM((n_ch, _CH), jnp.int32),
            pltpu.VMEM((b_per_w, d), jnp.float32),
            pltpu.SemaphoreType.DMA,
        ],
        compiler_params=pltpu.CompilerParams(use_tc_tiling_on_sc=False),
    )
    def gather_k(idx_hbm, table_hbm, out_hbm, idx_v, rows_v, sem):
        wid = lax.axis_index("s") * 2 + lax.axis_index("c")
        base = wid * b_per_w
        pltpu.sync_copy(idx_hbm.at[wid], idx_v)
        copies = []
        for k in range(n_ch):
            copies.append(pltpu.async_copy(
                table_hbm.at[idx_v.at[k]],
                rows_v.at[pl.ds(k * _CH, _CH)], sem))
        for cp in copies:
            cp.wait()
        pltpu.sync_copy(rows_v, out_hbm.at[pl.ds(base, b_per_w)])

    return gather_k


def kernel(x, embed):
    xf = x.astype(jnp.float32)
    b0, b1, d = xf.shape
    c = embed.shape[1]
    n = b0 * b1

    x_flat = xf.reshape(n, d)
    e = embed.reshape(c, d)

    idx = _vq_argmax(x_flat, e)

    idx3 = idx.reshape(_NW, (n // _NW) // _CH, _CH)
    quant = _make_sc_gather(n, c, d)(idx3, e)

    return quant.reshape(b0, b1, d), idx.reshape(b0, b1)


# speedup vs baseline: 3.4904x; 1.5107x over previous
import functools

import jax
import jax.numpy as jnp
from jax import lax
from jax.experimental import pallas as pl
from jax.experimental.pallas import tpu as pltpu
from jax.experimental.pallas import tpu_sc as plsc


_TN = 512
_TSUB = 1024


def _fold_tree(vals, idxs):
    while len(vals) > 1:
        nv, ni = [], []
        for p in range(0, len(vals), 2):
            vl, vr = vals[p], vals[p + 1]
            il, ir = idxs[p], idxs[p + 1]
            take_r = vr > vl
            nv.append(jnp.maximum(vl, vr))
            ni.append(jnp.where(take_r, ir, il))
        vals, idxs = nv, ni
    return vals[0], idxs[0]


def _argmax_body(x_ref, e_ref, idx_ref, en_s):
    c = e_ref.shape[0]

    @pl.when(pl.program_id(0) == 0)
    def _():
        e = e_ref[...]
        en_s[...] = e / jnp.clip(
            jnp.sqrt(jnp.sum(e * e, axis=-1, keepdims=True)), 1e-12, None)

    xb = x_ref[...]
    xn = xb / jnp.clip(
        jnp.sqrt(jnp.sum(xb * xb, axis=-1, keepdims=True)), 1e-12, None)

    pvals, pidxs = [], []
    for t in range(c // _TSUB):
        en = en_s[pl.ds(t * _TSUB, _TSUB), :]
        dist = lax.dot_general(
            xn, en, (((1,), (1,)), ((), ())),
            preferred_element_type=jnp.float32)
        vals = [dist[:, s * 128:(s + 1) * 128] for s in range(_TSUB // 128)]
        idxs = [jnp.full((_TN, 128), t * _TSUB + s * 128, jnp.int32)
                for s in range(_TSUB // 128)]
        v, ix = _fold_tree(vals, idxs)
        pvals.append(v)
        pidxs.append(ix)

    v, ix = _fold_tree(pvals, pidxs)
    rowmax = jnp.max(v, axis=-1)
    cidx = ix + lax.broadcasted_iota(jnp.int32, ix.shape, 1)
    cand = jnp.where(v == rowmax[:, None], cidx, jnp.int32(2**30))
    idx_ref[...] = jnp.min(cand, axis=-1).reshape(idx_ref.shape)


def _vq_argmax(x_flat, e):
    n, d = x_flat.shape
    c = e.shape[0]
    nb = n // _TN
    out = pl.pallas_call(
        _argmax_body,
        grid=(nb,),
        in_specs=[
            pl.BlockSpec((_TN, d), lambda i: (i, 0)),
            pl.BlockSpec((c, d), lambda i: (0, 0)),
        ],
        out_specs=pl.BlockSpec((1, 1, _TN), lambda i: (i, 0, 0)),
        out_shape=jax.ShapeDtypeStruct((nb, 1, _TN), jnp.int32),
        scratch_shapes=[
            pltpu.VMEM((c, d), jnp.float32),
        ],
        compiler_params=pltpu.CompilerParams(
            dimension_semantics=("arbitrary",)),
    )(x_flat, e)
    return out.reshape(n)


_NW = 32
_CH = 128


def _make_sc_gather(b, v, d):
    b_per_w = b // _NW
    n_ch = b_per_w // _CH
    mesh = plsc.VectorSubcoreMesh(core_axis_name="c", subcore_axis_name="s")

    @functools.partial(
        pl.kernel, mesh=mesh,
        out_type=jax.ShapeDtypeStruct((b, d), jnp.float32),
        scratch_types=[
            pltpu.VMEM((n_ch, _CH), jnp.int32),
            pltpu.VMEM((b_per_w, d), jnp.float32),
            pltpu.SemaphoreType.DMA,
        ],
        compiler_params=pltpu.CompilerParams(use_tc_tiling_on_sc=False),
    )
    def gather_k(idx_hbm, table_hbm, out_hbm, idx_v, rows_v, sem):
        wid = lax.axis_index("s") * 2 + lax.axis_index("c")
        base = wid * b_per_w
        pltpu.sync_copy(idx_hbm.at[wid], idx_v)
        copies = []
        for k in range(n_ch):
            copies.append(pltpu.async_copy(
                table_hbm.at[idx_v.at[k]],
                rows_v.at[pl.ds(k * _CH, _CH)], sem))
        for cp in copies:
            cp.wait()
        pltpu.sync_copy(rows_v, out_hbm.at[pl.ds(base, b_per_w)])

    return gather_k


def kernel(x, embed):
    xf = x.astype(jnp.float32)
    b0, b1, d = xf.shape
    c = embed.shape[1]
    n = b0 * b1

    x_flat = xf.reshape(n, d)
    e = embed.reshape(c, d)

    idx = _vq_argmax(x_flat, e)

    idx3 = idx.reshape(_NW, (n // _NW) // _CH, _CH)
    quant = _make_sc_gather(n, c, d)(idx3, e)

    return quant.reshape(b0, b1, d), idx.reshape(b0, b1)


# speedup vs baseline: 4.0918x; 1.1723x over previous
import functools

import jax
import jax.numpy as jnp
from jax import lax
from jax.experimental import pallas as pl
from jax.experimental.pallas import tpu as pltpu
from jax.experimental.pallas import tpu_sc as plsc


_TN = 1024
_TSUB = 1024


def _fold_tree(vals, idxs):
    while len(vals) > 1:
        nv, ni = [], []
        for p in range(0, len(vals), 2):
            vl, vr = vals[p], vals[p + 1]
            il, ir = idxs[p], idxs[p + 1]
            take_r = vr > vl
            nv.append(jnp.maximum(vl, vr))
            ni.append(jnp.where(take_r, ir, il))
        vals, idxs = nv, ni
    return vals[0], idxs[0]


def _argmax_body(x_ref, e_ref, idx_ref, en_s):
    c = e_ref.shape[0]

    @pl.when(pl.program_id(0) == 0)
    def _():
        e = e_ref[...]
        en_s[...] = e / jnp.clip(
            jnp.sqrt(jnp.sum(e * e, axis=-1, keepdims=True)), 1e-12, None)

    xb = x_ref[...]
    xn = xb / jnp.clip(
        jnp.sqrt(jnp.sum(xb * xb, axis=-1, keepdims=True)), 1e-12, None)

    pvals, pidxs = [], []
    for t in range(c // _TSUB):
        en = en_s[pl.ds(t * _TSUB, _TSUB), :]
        dist = lax.dot_general(
            xn, en, (((1,), (1,)), ((), ())),
            preferred_element_type=jnp.float32)
        vals = [dist[:, s * 128:(s + 1) * 128] for s in range(_TSUB // 128)]
        idxs = [jnp.full((_TN, 128), t * _TSUB + s * 128, jnp.int32)
                for s in range(_TSUB // 128)]
        v, ix = _fold_tree(vals, idxs)
        pvals.append(v)
        pidxs.append(ix)

    v, ix = _fold_tree(pvals, pidxs)
    rowmax = jnp.max(v, axis=-1)
    cidx = ix + lax.broadcasted_iota(jnp.int32, ix.shape, 1)
    cand = jnp.where(v == rowmax[:, None], cidx, jnp.int32(2**30))
    idx_ref[...] = jnp.min(cand, axis=-1).reshape(idx_ref.shape)


def _vq_argmax(x_flat, e):
    n, d = x_flat.shape
    c = e.shape[0]
    nb = n // _TN
    out = pl.pallas_call(
        _argmax_body,
        grid=(nb,),
        in_specs=[
            pl.BlockSpec((_TN, d), lambda i: (i, 0)),
            pl.BlockSpec((c, d), lambda i: (0, 0)),
        ],
        out_specs=pl.BlockSpec((1, 8, _TN // 8), lambda i: (i, 0, 0)),
        out_shape=jax.ShapeDtypeStruct((nb, 8, _TN // 8), jnp.int32),
        scratch_shapes=[
            pltpu.VMEM((c, d), jnp.float32),
        ],
        compiler_params=pltpu.CompilerParams(
            dimension_semantics=("arbitrary",)),
    )(x_flat, e)
    return out


_NW = 32
_CH = 128


def _make_sc_gather(b, v, d):
    b_per_w = b // _NW
    n_ch = b_per_w // _CH
    mesh = plsc.VectorSubcoreMesh(core_axis_name="c", subcore_axis_name="s")

    @functools.partial(
        pl.kernel, mesh=mesh,
        out_type=jax.ShapeDtypeStruct((b, d), jnp.float32),
        scratch_types=[
            pltpu.VMEM((n_ch, _CH), jnp.int32),
            pltpu.VMEM((b_per_w, d), jnp.float32),
            pltpu.SemaphoreType.DMA,
        ],
        compiler_params=pltpu.CompilerParams(use_tc_tiling_on_sc=False),
    )
    def gather_k(idx_hbm, table_hbm, out_hbm, idx_v, rows_v, sem):
        wid = lax.axis_index("s") * 2 + lax.axis_index("c")
        base = wid * b_per_w
        pltpu.sync_copy(idx_hbm.at[wid], idx_v)
        copies = []
        for k in range(n_ch):
            copies.append(pltpu.async_copy(
                table_hbm.at[idx_v.at[k]],
                rows_v.at[pl.ds(k * _CH, _CH)], sem))
        for cp in copies:
            cp.wait()
        pltpu.sync_copy(rows_v, out_hbm.at[pl.ds(base, b_per_w)])

    return gather_k


def kernel(x, embed):
    xf = x.astype(jnp.float32)
    b0, b1, d = xf.shape
    c = embed.shape[1]
    n = b0 * b1

    x_flat = xf.reshape(n, d)
    e = embed.reshape(c, d)

    idx_tiles = _vq_argmax(x_flat, e)

    quant = _make_sc_gather(n, c, d)(idx_tiles, e)

    return quant.reshape(b0, b1, d), idx_tiles.reshape(n).reshape(b0, b1)


# speedup vs baseline: 4.1786x; 1.0212x over previous
import functools

import jax
import jax.numpy as jnp
from jax import lax
from jax.experimental import pallas as pl
from jax.experimental.pallas import tpu as pltpu
from jax.experimental.pallas import tpu_sc as plsc


_TN = 2048
_TSUB = 1024


def _fold_tree(vals, idxs):
    while len(vals) > 1:
        nv, ni = [], []
        for p in range(0, len(vals), 2):
            vl, vr = vals[p], vals[p + 1]
            il, ir = idxs[p], idxs[p + 1]
            take_r = vr > vl
            nv.append(jnp.maximum(vl, vr))
            ni.append(jnp.where(take_r, ir, il))
        vals, idxs = nv, ni
    return vals[0], idxs[0]


def _argmax_body(x_ref, e_ref, idx_ref, en_s):
    c = e_ref.shape[0]

    @pl.when(pl.program_id(0) == 0)
    def _():
        e = e_ref[...]
        en_s[...] = e / jnp.clip(
            jnp.sqrt(jnp.sum(e * e, axis=-1, keepdims=True)), 1e-12, None)

    xb = x_ref[...]
    xn = xb / jnp.clip(
        jnp.sqrt(jnp.sum(xb * xb, axis=-1, keepdims=True)), 1e-12, None)

    pvals, pidxs = [], []
    for t in range(c // _TSUB):
        en = en_s[pl.ds(t * _TSUB, _TSUB), :]
        dist = lax.dot_general(
            xn, en, (((1,), (1,)), ((), ())),
            preferred_element_type=jnp.float32)
        vals = [dist[:, s * 128:(s + 1) * 128] for s in range(_TSUB // 128)]
        idxs = [jnp.full((_TN, 128), t * _TSUB + s * 128, jnp.int32)
                for s in range(_TSUB // 128)]
        v, ix = _fold_tree(vals, idxs)
        pvals.append(v)
        pidxs.append(ix)

    v, ix = _fold_tree(pvals, pidxs)
    rowmax = jnp.max(v, axis=-1)
    cidx = ix + lax.broadcasted_iota(jnp.int32, ix.shape, 1)
    cand = jnp.where(v == rowmax[:, None], cidx, jnp.int32(2**30))
    idx_ref[...] = jnp.min(cand, axis=-1).reshape(idx_ref.shape)


def _vq_argmax(x_flat, e):
    n, d = x_flat.shape
    c = e.shape[0]
    nb = n // _TN
    out = pl.pallas_call(
        _argmax_body,
        grid=(nb,),
        in_specs=[
            pl.BlockSpec((_TN, d), lambda i: (i, 0)),
            pl.BlockSpec((c, d), lambda i: (0, 0)),
        ],
        out_specs=pl.BlockSpec((_TN // 1024, 8, 128), lambda i: (i, 0, 0)),
        out_shape=jax.ShapeDtypeStruct((nb * (_TN // 1024), 8, 128), jnp.int32),
        scratch_shapes=[
            pltpu.VMEM((c, d), jnp.float32),
        ],
        compiler_params=pltpu.CompilerParams(
            dimension_semantics=("arbitrary",)),
    )(x_flat, e)
    return out


_NW = 32
_CH = 128


def _make_sc_gather(b, v, d):
    b_per_w = b // _NW
    n_ch = b_per_w // _CH
    mesh = plsc.VectorSubcoreMesh(core_axis_name="c", subcore_axis_name="s")

    @functools.partial(
        pl.kernel, mesh=mesh,
        out_type=jax.ShapeDtypeStruct((b, d), jnp.float32),
        scratch_types=[
            pltpu.VMEM((n_ch, _CH), jnp.int32),
            pltpu.VMEM((b_per_w, d), jnp.float32),
            pltpu.SemaphoreType.DMA,
        ],
        compiler_params=pltpu.CompilerParams(use_tc_tiling_on_sc=False),
    )
    def gather_k(idx_hbm, table_hbm, out_hbm, idx_v, rows_v, sem):
        wid = lax.axis_index("s") * 2 + lax.axis_index("c")
        base = wid * b_per_w
        pltpu.sync_copy(idx_hbm.at[wid], idx_v)
        copies = []
        for k in range(n_ch):
            copies.append(pltpu.async_copy(
                table_hbm.at[idx_v.at[k]],
                rows_v.at[pl.ds(k * _CH, _CH)], sem))
        for cp in copies:
            cp.wait()
        pltpu.sync_copy(rows_v, out_hbm.at[pl.ds(base, b_per_w)])

    return gather_k


def kernel(x, embed):
    xf = x.astype(jnp.float32)
    b0, b1, d = xf.shape
    c = embed.shape[1]
    n = b0 * b1

    x_flat = xf.reshape(n, d)
    e = embed.reshape(c, d)

    idx_tiles = _vq_argmax(x_flat, e)

    quant = _make_sc_gather(n, c, d)(idx_tiles, e)

    return quant.reshape(b0, b1, d), idx_tiles.reshape(n).reshape(b0, b1)
